# baseline (speedup 1.0000x reference)
import jax
import jax.numpy as jnp
from jax import lax
from jax.experimental import pallas as pl
from jax.experimental.pallas import tpu as pltpu
from jax.experimental.pallas import tpu_sc as plsc

N = 10000
E = 320000
D = 128

NC = 2
NS = 16
L = 16
NW = NC * NS

EPW = E // NW
K = 128
CH = 79
EPAD = CH * K
NP = 10240
DUMP = NP - 1
RPT = NP // NS

BLK = 5000
GRID = N // BLK

_MESH = plsc.VectorSubcoreMesh(core_axis_name="c", subcore_axis_name="s")


def _deg_body(row3_hbm, degp_hbm, rowt, ones, zbufh, hist, sem):
    cid = lax.axis_index("c")
    sid = lax.axis_index("s")
    wid = cid * NS + sid
    rbase = sid * RPT
    zeros16 = jnp.zeros((L,), jnp.float32)
    ones16 = jnp.ones((L,), jnp.float32)

    pltpu.sync_copy(row3_hbm.at[wid], rowt)

    def fill(i, _):
        ones[i, :] = ones16
        return 0

    lax.fori_loop(0, K, fill, 0)

    def fillz(i, _):
        zbufh[i, :] = zeros16
        return 0

    lax.fori_loop(0, K, fillz, 0)
    for k in range(RPT // K):
        pltpu.sync_copy(zbufh, hist.at[pl.ds(rbase + k * K, K)])
    plsc.subcore_barrier()

    def scat(c):
        pltpu.make_async_copy(ones, hist.at[rowt.at[c]], sem).start(add=True)

    def drain():
        pltpu.make_async_copy(ones, hist.at[rowt.at[0]], sem).wait()

    for c in range(4):
        scat(c)

    def body(c, _):
        scat(c)
        drain()
        return 0

    lax.fori_loop(4, CH, body, 0)
    for _ in range(4):
        drain()
    plsc.subcore_barrier()
    for k in range(RPT // K):
        pltpu.sync_copy(hist.at[pl.ds(rbase + k * K, K)], zbufh)
        pltpu.sync_copy(zbufh, degp_hbm.at[cid, pl.ds(rbase + k * K, K)])


_deg_call = pl.kernel(
    _deg_body,
    out_type=jax.ShapeDtypeStruct((NC, NP, L), jnp.float32),
    mesh=_MESH,
    scratch_types=[
        pltpu.VMEM((CH, K), jnp.int32),
        pltpu.VMEM((K, L), jnp.float32),
        pltpu.VMEM((K, L), jnp.float32),
        pltpu.VMEM_SHARED((NP, L), jnp.float32),
        pltpu.SemaphoreType.DMA,
    ],
)


KA = 80
CA = EPW // KA


def _agg_body(hs_hbm, col2_hbm, rowa_hbm, part_hbm,
              colv, rowv, rows0, rows1, acc, gsem0, gsem1):
    cid = lax.axis_index("c")
    sid = lax.axis_index("s")
    wid = cid * NS + sid
    rbase = sid * RPT
    zeros16 = jnp.zeros((L,), jnp.float32)

    pltpu.sync_copy(col2_hbm.at[wid], colv)
    pltpu.sync_copy(rowa_hbm.at[wid], rowv)

    def zb(i, _):
        rows0[i // 8, pl.ds((i % 8) * L, L)] = zeros16
        return 0

    lax.fori_loop(0, KA * 8, zb, 0)
    for k in range(RPT // KA):
        pltpu.sync_copy(rows0, acc.at[pl.ds(rbase + k * KA, KA)])
    plsc.subcore_barrier()

    rows = (rows0, rows1)
    gsem = (gsem0, gsem1)

    def g_start(c, b):
        pltpu.make_async_copy(
            hs_hbm.at[colv.at[pl.ds(c * KA, KA)]], rows[b], gsem[b]).start()

    def g_wait(c, b):
        pltpu.make_async_copy(
            hs_hbm.at[colv.at[pl.ds(c * KA, KA)]], rows[b], gsem[b]).wait()

    def scat(c, b):
        pltpu.sync_copy(rows[b], acc.at[rowv.at[c]], add=True)

    g_start(0, 0)

    def body(i, _):
        c = 2 * i
        g_start(c + 1, 1)
        g_wait(c, 0)
        scat(c, 0)
        g_start(c + 2, 0)
        g_wait(c + 1, 1)
        scat(c + 1, 1)
        return 0

    lax.fori_loop(0, (CA - 1) // 2, body, 0)
    g_wait(CA - 1, 0)
    scat(CA - 1, 0)
    plsc.subcore_barrier()
    for k in range(RPT // KA):
        pltpu.sync_copy(acc.at[pl.ds(rbase + k * KA, KA)], rows0)
        pltpu.sync_copy(rows0, part_hbm.at[cid, pl.ds(rbase + k * KA, KA)])


_agg_call = pl.kernel(
    _agg_body,
    out_type=jax.ShapeDtypeStruct((NC, NP, D), jnp.float32),
    mesh=_MESH,
    scratch_types=[
        pltpu.VMEM((EPW,), jnp.int32),
        pltpu.VMEM((CA, KA), jnp.int32),
        pltpu.VMEM((KA, D), jnp.float32),
        pltpu.VMEM((KA, D), jnp.float32),
        pltpu.VMEM_SHARED((NP, D), jnp.float32),
        pltpu.SemaphoreType.DMA,
        pltpu.SemaphoreType.DMA,
    ],
)


def _dinv_of(dp):
    return lax.rsqrt(1.0 + dp[0, :, 0:1] + dp[1, :, 0:1])


def _tc1_body(x_ref, w_ref, dp_ref, out_ref):
    dinv = _dinv_of(dp_ref[...])
    h = jnp.dot(x_ref[...], w_ref[...], preferred_element_type=jnp.float32)
    out_ref[...] = h * dinv


def _tc2_body(hs_ref, p_ref, dp_ref, w_ref, out_ref):
    dinv = _dinv_of(dp_ref[...])
    p = p_ref[...]
    h1 = jnp.maximum((hs_ref[...] + p[0] + p[1]) * dinv, 0.0)
    out_ref[...] = jnp.dot(h1, w_ref[...],
                           preferred_element_type=jnp.float32) * dinv


def _tc3_body(hs_ref, p_ref, dp_ref, out_ref):
    dinv = _dinv_of(dp_ref[...])
    p = p_ref[...]
    t = jnp.maximum((hs_ref[...] + p[0] + p[1]) * dinv, 0.0)
    m = jnp.max(t, axis=-1, keepdims=True)
    e = jnp.exp(t - m)
    out_ref[...] = e / jnp.sum(e, axis=-1, keepdims=True)


_ROWB = pl.BlockSpec((BLK, D), lambda i: (i, 0))
_WB = pl.BlockSpec((D, D), lambda i: (0, 0))
_DPB = pl.BlockSpec((NC, BLK, L), lambda i: (0, i, 0))
_PB = pl.BlockSpec((NC, BLK, D), lambda i: (0, i, 0))
_OUT = jax.ShapeDtypeStruct((N, D), jnp.float32)

_tc1 = pl.pallas_call(
    _tc1_body, grid=(GRID,),
    in_specs=[_ROWB, _WB, _DPB], out_specs=_ROWB, out_shape=_OUT)

_tc2 = pl.pallas_call(
    _tc2_body, grid=(GRID,),
    in_specs=[_ROWB, _PB, _DPB, _WB], out_specs=_ROWB, out_shape=_OUT)

_tc3 = pl.pallas_call(
    _tc3_body, grid=(GRID,),
    in_specs=[_ROWB, _PB, _DPB], out_specs=_ROWB, out_shape=_OUT)


@jax.jit
def kernel(x, edge_index, W0, W1):
    row = edge_index[0].reshape(NW, EPW)
    col2 = edge_index[1].reshape(NW, EPW)
    pad_r = jnp.full((NW, EPAD - EPW), DUMP, jnp.int32)
    row3 = jnp.concatenate([row, pad_r], axis=1).reshape(NW, CH, K)
    rowa = row.reshape(NW, CA, KA)
    degp = _deg_call(row3)
    hs0 = _tc1(x, W0, degp)
    p1 = _agg_call(hs0, col2, rowa)
    hs1 = _tc2(hs0, p1, degp, W1)
    p2 = _agg_call(hs1, col2, rowa)
    return _tc3(hs1, p2, degp)


# speedup vs baseline: 1.0351x; 1.0351x over previous
import jax
import jax.numpy as jnp
from jax import lax
from jax.experimental import pallas as pl
from jax.experimental.pallas import tpu as pltpu
from jax.experimental.pallas import tpu_sc as plsc

N = 10000
E = 320000
D = 128

NC = 2
NS = 16
L = 16
NW = NC * NS

EPW = E // NW
K = 128
CH = 79
EPAD = CH * K
NP = 10240
DUMP = NP - 1
RPT = NP // NS

BLK = 5000
GRID = N // BLK

_MESH = plsc.VectorSubcoreMesh(core_axis_name="c", subcore_axis_name="s")


def _deg_body(row3_hbm, degp_hbm, rowt, ones, zbufh, hist, sem):
    cid = lax.axis_index("c")
    sid = lax.axis_index("s")
    wid = cid * NS + sid
    rbase = sid * RPT
    zeros16 = jnp.zeros((L,), jnp.float32)
    ones16 = jnp.ones((L,), jnp.float32)

    pltpu.sync_copy(row3_hbm.at[wid], rowt)

    def fill(i, _):
        ones[i, :] = ones16
        return 0

    lax.fori_loop(0, K, fill, 0)

    def fillz(i, _):
        zbufh[i, :] = zeros16
        return 0

    lax.fori_loop(0, K, fillz, 0)
    for k in range(RPT // K):
        pltpu.sync_copy(zbufh, hist.at[pl.ds(rbase + k * K, K)])
    plsc.subcore_barrier()

    def scat(c):
        pltpu.make_async_copy(ones, hist.at[rowt.at[c]], sem).start(add=True)

    def drain():
        pltpu.make_async_copy(ones, hist.at[rowt.at[0]], sem).wait()

    for c in range(4):
        scat(c)

    def body(c, _):
        scat(c)
        drain()
        return 0

    lax.fori_loop(4, CH, body, 0)
    for _ in range(4):
        drain()
    plsc.subcore_barrier()
    for k in range(RPT // K):
        pltpu.sync_copy(hist.at[pl.ds(rbase + k * K, K)], zbufh)
        pltpu.sync_copy(zbufh, degp_hbm.at[cid, pl.ds(rbase + k * K, K)])


_deg_call = pl.kernel(
    _deg_body,
    out_type=jax.ShapeDtypeStruct((NC, NP, L), jnp.float32),
    mesh=_MESH,
    scratch_types=[
        pltpu.VMEM((CH, K), jnp.int32),
        pltpu.VMEM((K, L), jnp.float32),
        pltpu.VMEM((K, L), jnp.float32),
        pltpu.VMEM_SHARED((NP, L), jnp.float32),
        pltpu.SemaphoreType.DMA,
    ],
)


KA = 80
CA = EPW // KA


def _agg_body(hs_hbm, col2_hbm, rowa_hbm, part_hbm,
              colv, rowv, rows0, rows1, acc, gsem0, gsem1):
    cid = lax.axis_index("c")
    sid = lax.axis_index("s")
    wid = cid * NS + sid
    rbase = sid * RPT
    zeros16 = jnp.zeros((L,), jnp.float32)

    pltpu.make_async_copy(col2_hbm.at[wid], colv, gsem0).start()
    pltpu.make_async_copy(rowa_hbm.at[wid], rowv, gsem1).start()

    def zb(i, _):
        rows0[i // 8, pl.ds((i % 8) * L, L)] = zeros16
        return 0

    lax.fori_loop(0, KA * 8, zb, 0)
    for k in range(RPT // KA):
        pltpu.make_async_copy(
            rows0, acc.at[pl.ds(rbase + k * KA, KA)], gsem0).start()
    for k in range(RPT // KA):
        pltpu.make_async_copy(
            rows0, acc.at[pl.ds(rbase + k * KA, KA)], gsem0).wait()
    pltpu.make_async_copy(col2_hbm.at[wid], colv, gsem0).wait()
    pltpu.make_async_copy(rowa_hbm.at[wid], rowv, gsem1).wait()
    plsc.subcore_barrier()

    rows = (rows0, rows1)
    gsem = (gsem0, gsem1)

    def g_start(c, b):
        pltpu.make_async_copy(
            hs_hbm.at[colv.at[pl.ds(c * KA, KA)]], rows[b], gsem[b]).start()

    def g_wait(c, b):
        pltpu.make_async_copy(
            hs_hbm.at[colv.at[pl.ds(c * KA, KA)]], rows[b], gsem[b]).wait()

    def scat(c, b):
        pltpu.sync_copy(rows[b], acc.at[rowv.at[c]], add=True)

    g_start(0, 0)

    def body(i, _):
        c = 2 * i
        g_start(c + 1, 1)
        g_wait(c, 0)
        scat(c, 0)
        g_start(c + 2, 0)
        g_wait(c + 1, 1)
        scat(c + 1, 1)
        return 0

    lax.fori_loop(0, (CA - 1) // 2, body, 0)
    g_wait(CA - 1, 0)
    scat(CA - 1, 0)
    plsc.subcore_barrier()

    def w_start(k, b):
        pltpu.make_async_copy(
            rows[b], part_hbm.at[cid, pl.ds(rbase + k * KA, KA)],
            gsem[b]).start()

    def w_wait(k, b):
        pltpu.make_async_copy(
            rows[b], part_hbm.at[cid, pl.ds(rbase + k * KA, KA)],
            gsem[b]).wait()

    for k in range(RPT // KA):
        b = k % 2
        if k >= 2:
            w_wait(k - 2, b)
        pltpu.sync_copy(acc.at[pl.ds(rbase + k * KA, KA)], rows[b])
        w_start(k, b)
    w_wait(RPT // KA - 2, 0)
    w_wait(RPT // KA - 1, 1)


_agg_call = pl.kernel(
    _agg_body,
    out_type=jax.ShapeDtypeStruct((NC, NP, D), jnp.float32),
    mesh=_MESH,
    scratch_types=[
        pltpu.VMEM((EPW,), jnp.int32),
        pltpu.VMEM((CA, KA), jnp.int32),
        pltpu.VMEM((KA, D), jnp.float32),
        pltpu.VMEM((KA, D), jnp.float32),
        pltpu.VMEM_SHARED((NP, D), jnp.float32),
        pltpu.SemaphoreType.DMA,
        pltpu.SemaphoreType.DMA,
    ],
)


def _dinv_of(dp):
    return lax.rsqrt(1.0 + dp[0, :, 0:1] + dp[1, :, 0:1])


def _tc1_body(x_ref, w_ref, dp_ref, out_ref):
    dinv = _dinv_of(dp_ref[...])
    h = jnp.dot(x_ref[...], w_ref[...], preferred_element_type=jnp.float32)
    out_ref[...] = h * dinv


def _tc2_body(hs_ref, p_ref, dp_ref, w_ref, out_ref):
    dinv = _dinv_of(dp_ref[...])
    p = p_ref[...]
    h1 = jnp.maximum((hs_ref[...] + p[0] + p[1]) * dinv, 0.0)
    out_ref[...] = jnp.dot(h1, w_ref[...],
                           preferred_element_type=jnp.float32) * dinv


def _tc3_body(hs_ref, p_ref, dp_ref, out_ref):
    dinv = _dinv_of(dp_ref[...])
    p = p_ref[...]
    t = jnp.maximum((hs_ref[...] + p[0] + p[1]) * dinv, 0.0)
    m = jnp.max(t, axis=-1, keepdims=True)
    e = jnp.exp(t - m)
    out_ref[...] = e / jnp.sum(e, axis=-1, keepdims=True)


_ROWB = pl.BlockSpec((BLK, D), lambda i: (i, 0))
_WB = pl.BlockSpec((D, D), lambda i: (0, 0))
_DPB = pl.BlockSpec((NC, BLK, L), lambda i: (0, i, 0))
_PB = pl.BlockSpec((NC, BLK, D), lambda i: (0, i, 0))
_OUT = jax.ShapeDtypeStruct((N, D), jnp.float32)

_tc1 = pl.pallas_call(
    _tc1_body, grid=(GRID,),
    in_specs=[_ROWB, _WB, _DPB], out_specs=_ROWB, out_shape=_OUT)

_tc2 = pl.pallas_call(
    _tc2_body, grid=(GRID,),
    in_specs=[_ROWB, _PB, _DPB, _WB], out_specs=_ROWB, out_shape=_OUT)

_tc3 = pl.pallas_call(
    _tc3_body, grid=(GRID,),
    in_specs=[_ROWB, _PB, _DPB], out_specs=_ROWB, out_shape=_OUT)


@jax.jit
def kernel(x, edge_index, W0, W1):
    row = edge_index[0].reshape(NW, EPW)
    col2 = edge_index[1].reshape(NW, EPW)
    pad_r = jnp.full((NW, EPAD - EPW), DUMP, jnp.int32)
    row3 = jnp.concatenate([row, pad_r], axis=1).reshape(NW, CH, K)
    rowa = row.reshape(NW, CA, KA)
    degp = _deg_call(row3)
    hs0 = _tc1(x, W0, degp)
    p1 = _agg_call(hs0, col2, rowa)
    hs1 = _tc2(hs0, p1, degp, W1)
    p2 = _agg_call(hs1, col2, rowa)
    return _tc3(hs1, p2, degp)
